# baseline (device time: 92752 ns/iter reference)
import jax
import jax.numpy as jnp
from jax import lax
from jax.experimental import pallas as pl
from jax.experimental.pallas import tpu as pltpu

N_DEV = 8
B, SQ, SKV, DM = 2, 512, 512, 768
HQ_PER = 8
DH = 64
DP = HQ_PER * DH
BLK = 64
ROWS = B * SQ
HALF = ROWS // 2
CHUNK = HALF // N_DEV


def kernel(x, Wq, K_ext, V_ext, Wo):
    my = lax.axis_index("i")
    Kh = lax.dynamic_slice_in_dim(K_ext, my * HQ_PER, HQ_PER, axis=2)
    Vh = lax.dynamic_slice_in_dim(V_ext, my * HQ_PER, HQ_PER, axis=2)
    Kt = jnp.transpose(Kh, (0, 2, 1, 3))
    Vt = jnp.transpose(Vh, (0, 2, 1, 3))
    xf = x.reshape(ROWS, DM)

    def body(x_ref, wq_ref, k_ref, v_ref, wo_ref, out_ref,
             q_ref, comm_r, comm_l,
             send_sems_r, recv_sems_r, send_sems_l, recv_sems_l):
        my_pos = lax.axis_index("i")
        left = (my_pos - 1) % N_DEV
        right = (my_pos + 1) % N_DEV

        barrier_sem = pltpu.get_barrier_semaphore()
        for nbr in (left, right):
            pl.semaphore_signal(
                barrier_sem, inc=1,
                device_id=(nbr,), device_id_type=pl.DeviceIdType.MESH,
            )
        pl.semaphore_wait(barrier_sem, 2)

        q_ref[...] = jnp.dot(x_ref[...], wq_ref[...],
                             preferred_element_type=jnp.float32)

        kb = lax.broadcasted_iota(jnp.int32, (CHUNK, SKV), 1) // BLK

        def compute_chunk(b, c):
            q64 = q_ref[pl.ds(b * SQ + c * CHUNK, CHUNK), :]
            parts = []
            for h in range(HQ_PER):
                s = lax.dot_general(
                    q64[:, h * DH:(h + 1) * DH], k_ref[b, h],
                    (((1,), (1,)), ((), ())),
                    preferred_element_type=jnp.float32) * 0.125
                s = jnp.where(kb <= c, s, -1e9)
                m = jnp.max(s, axis=-1, keepdims=True)
                w = jnp.exp(s - m)
                w = w / jnp.sum(w, axis=-1, keepdims=True)
                parts.append(jnp.dot(w, v_ref[b, h],
                                     preferred_element_type=jnp.float32))
            ctx64 = jnp.concatenate(parts, axis=1)
            out_ref[pl.ds(b * SQ + c * CHUNK, CHUNK), :] = jnp.dot(
                ctx64, wo_ref[...], preferred_element_type=jnp.float32)

        compute_chunk(0, my_pos)
        compute_chunk(1, my_pos)

        for t in range(N_DEV - 1):
            sc_r = (my_pos - t) % N_DEV
            rc_r = (my_pos - t - 1) % N_DEV
            sc_l = (my_pos + t) % N_DEV
            rc_l = (my_pos + t + 1) % N_DEV
            slot = t % 2
            rdma_r = pltpu.make_async_remote_copy(
                src_ref=out_ref.at[pl.ds(sc_r * CHUNK, CHUNK), :],
                dst_ref=comm_r.at[slot],
                send_sem=send_sems_r.at[slot],
                recv_sem=recv_sems_r.at[slot],
                device_id=(right,),
                device_id_type=pl.DeviceIdType.MESH,
            )
            rdma_l = pltpu.make_async_remote_copy(
                src_ref=out_ref.at[pl.ds(HALF + sc_l * CHUNK, CHUNK), :],
                dst_ref=comm_l.at[slot],
                send_sem=send_sems_l.at[slot],
                recv_sem=recv_sems_l.at[slot],
                device_id=(left,),
                device_id_type=pl.DeviceIdType.MESH,
            )
            rdma_r.start()
            rdma_l.start()
            compute_chunk(0, rc_r)
            compute_chunk(1, rc_l)
            rdma_r.wait()
            out_ref[pl.ds(rc_r * CHUNK, CHUNK), :] = (
                out_ref[pl.ds(rc_r * CHUNK, CHUNK), :] + comm_r[slot])
            rdma_l.wait()
            out_ref[pl.ds(HALF + rc_l * CHUNK, CHUNK), :] = (
                out_ref[pl.ds(HALF + rc_l * CHUNK, CHUNK), :] + comm_l[slot])

        for t in range(N_DEV - 1):
            ac_r = (my_pos + 1 - t) % N_DEV
            ac_l = (my_pos - 1 + t) % N_DEV
            slot = t % 2
            rdma_r = pltpu.make_async_remote_copy(
                src_ref=out_ref.at[pl.ds(ac_r * CHUNK, CHUNK), :],
                dst_ref=out_ref.at[pl.ds(ac_r * CHUNK, CHUNK), :],
                send_sem=send_sems_r.at[slot],
                recv_sem=recv_sems_r.at[slot],
                device_id=(right,),
                device_id_type=pl.DeviceIdType.MESH,
            )
            rdma_l = pltpu.make_async_remote_copy(
                src_ref=out_ref.at[pl.ds(HALF + ac_l * CHUNK, CHUNK), :],
                dst_ref=out_ref.at[pl.ds(HALF + ac_l * CHUNK, CHUNK), :],
                send_sem=send_sems_l.at[slot],
                recv_sem=recv_sems_l.at[slot],
                device_id=(left,),
                device_id_type=pl.DeviceIdType.MESH,
            )
            rdma_r.start()
            rdma_l.start()
            rdma_r.wait()
            rdma_l.wait()

    out2d = pl.pallas_call(
        body,
        out_shape=jax.ShapeDtypeStruct((ROWS, DM), jnp.float32),
        in_specs=[pl.BlockSpec(memory_space=pltpu.VMEM)] * 5,
        out_specs=pl.BlockSpec(memory_space=pltpu.VMEM),
        scratch_shapes=[
            pltpu.VMEM((ROWS, DP), jnp.float32),
            pltpu.VMEM((2, CHUNK, DM), jnp.float32),
            pltpu.VMEM((2, CHUNK, DM), jnp.float32),
            pltpu.SemaphoreType.DMA((2,)),
            pltpu.SemaphoreType.DMA((2,)),
            pltpu.SemaphoreType.DMA((2,)),
            pltpu.SemaphoreType.DMA((2,)),
        ],
        compiler_params=pltpu.CompilerParams(collective_id=0),
    )(xf, Wq, Kt, Vt, Wo)
    return out2d.reshape(B, SQ, DM)


# device time: 55927 ns/iter; 1.6584x vs baseline; 1.6584x over previous
import jax
import jax.numpy as jnp
from jax import lax
from jax.experimental import pallas as pl
from jax.experimental.pallas import tpu as pltpu

N_DEV = 8
B, SQ, SKV, DM = 2, 512, 512, 768
HQ_PER = 8
DH = 64
DP = HQ_PER * DH
BLK = 64
ROWS = B * SQ

PARTS = (
    {"base": 0, "size": 384, "order": "xyz"},
    {"base": 384, "size": 320, "order": "yzx"},
    {"base": 704, "size": 320, "order": "zxy"},
)


def kernel(x, Wq, K_ext, V_ext, Wo):
    my = lax.axis_index("i")
    Kh = lax.dynamic_slice_in_dim(K_ext, my * HQ_PER, HQ_PER, axis=2)
    Vh = lax.dynamic_slice_in_dim(V_ext, my * HQ_PER, HQ_PER, axis=2)
    Kt = jnp.transpose(Kh, (0, 2, 1, 3))
    Vt = jnp.transpose(Vh, (0, 2, 1, 3))
    xf = x.reshape(ROWS, DM)

    def body(x_ref, wq_ref, k_ref, v_ref, wo_ref, out_ref,
             ctx_ref, comm_a, comm_b, comm_c,
             send_sems, recv_sems):
        my_pos = lax.axis_index("i")
        m4 = my_pos % 4
        bit = {
            "x": jnp.where((m4 == 1) | (m4 == 2), 1, 0),
            "y": jnp.where(m4 >= 2, 1, 0),
            "z": jnp.where(my_pos >= 4, 1, 0),
        }
        ptn = {
            "x": my_pos + 1 - 2 * (my_pos % 2),
            "y": 4 * (my_pos // 4) + 3 - m4,
            "z": (my_pos + 4) % N_DEV,
        }

        barrier_sem = pltpu.get_barrier_semaphore()
        for d in "xyz":
            pl.semaphore_signal(
                barrier_sem, inc=1,
                device_id=(ptn[d],), device_id_type=pl.DeviceIdType.MESH,
            )
        pl.semaphore_wait(barrier_sem, 3)

        qb = lax.broadcasted_iota(jnp.int32, (SQ, SKV), 0) // BLK
        kb = lax.broadcasted_iota(jnp.int32, (SQ, SKV), 1) // BLK
        mask = kb <= qb

        q_all = jnp.dot(x_ref[...], wq_ref[...],
                        preferred_element_type=jnp.float32)
        for b in range(B):
            for h in range(HQ_PER):
                q = q_all[b * SQ:(b + 1) * SQ, h * DH:(h + 1) * DH]
                s = lax.dot_general(
                    q, k_ref[b, h], (((1,), (1,)), ((), ())),
                    preferred_element_type=jnp.float32) * 0.125
                s = jnp.where(mask, s, -1e9)
                m = jnp.max(s, axis=-1, keepdims=True)
                w = jnp.exp(s - m)
                w = w / jnp.sum(w, axis=-1, keepdims=True)
                ctx_ref[b * SQ:(b + 1) * SQ, h * DH:(h + 1) * DH] = jnp.dot(
                    w, v_ref[b, h], preferred_element_type=jnp.float32)
        out_ref[...] = jnp.dot(ctx_ref[...], wo_ref[...],
                               preferred_element_type=jnp.float32)

        comms = {0: comm_a, 1: comm_b, 2: comm_c}
        cur_off = [jnp.int32(p["base"]) for p in PARTS]

        for k in range(3):
            slot = k % 2
            started = []
            for pi, p in enumerate(PARTS):
                h = (p["size"] >> k) // 2
                d = p["order"][k]
                send_off = cur_off[pi] + (1 - bit[d]) * h
                keep_off = cur_off[pi] + bit[d] * h
                hmax = p["size"] // 2
                rdma = pltpu.make_async_remote_copy(
                    src_ref=out_ref.at[pl.ds(send_off, h), :],
                    dst_ref=comms[pi].at[pl.ds(slot * hmax, h), :],
                    send_sem=send_sems.at[2 * pi + slot],
                    recv_sem=recv_sems.at[2 * pi + slot],
                    device_id=(ptn[d],),
                    device_id_type=pl.DeviceIdType.MESH,
                )
                rdma.start()
                started.append((rdma, pi, keep_off, h, slot * hmax))
                cur_off[pi] = keep_off
            for rdma, pi, keep_off, h, coff in started:
                rdma.wait()
                out_ref[pl.ds(keep_off, h), :] = (
                    out_ref[pl.ds(keep_off, h), :]
                    + comms[pi][pl.ds(coff, h), :])

        for k in range(3):
            slot = (3 + k) % 2
            started = []
            for pi, p in enumerate(PARTS):
                g = (p["size"] >> 3) << k
                d = p["order"][2 - k]
                rdma = pltpu.make_async_remote_copy(
                    src_ref=out_ref.at[pl.ds(cur_off[pi], g), :],
                    dst_ref=out_ref.at[pl.ds(cur_off[pi], g), :],
                    send_sem=send_sems.at[2 * pi + slot],
                    recv_sem=recv_sems.at[2 * pi + slot],
                    device_id=(ptn[d],),
                    device_id_type=pl.DeviceIdType.MESH,
                )
                rdma.start()
                started.append(rdma)
                cur_off[pi] = cur_off[pi] - bit[d] * g
            for rdma in started:
                rdma.wait()

    out2d = pl.pallas_call(
        body,
        out_shape=jax.ShapeDtypeStruct((ROWS, DM), jnp.float32),
        in_specs=[pl.BlockSpec(memory_space=pltpu.VMEM)] * 5,
        out_specs=pl.BlockSpec(memory_space=pltpu.VMEM),
        scratch_shapes=[
            pltpu.VMEM((ROWS, DP), jnp.float32),
            pltpu.VMEM((2 * 192, DM), jnp.float32),
            pltpu.VMEM((2 * 160, DM), jnp.float32),
            pltpu.VMEM((2 * 160, DM), jnp.float32),
            pltpu.SemaphoreType.DMA((6,)),
            pltpu.SemaphoreType.DMA((6,)),
        ],
        compiler_params=pltpu.CompilerParams(collective_id=0),
    )(xf, Wq, Kt, Vt, Wo)
    return out2d.reshape(B, SQ, DM)


# device time: 44102 ns/iter; 2.1031x vs baseline; 1.2681x over previous
import jax
import jax.numpy as jnp
from jax import lax
from jax.experimental import pallas as pl
from jax.experimental.pallas import tpu as pltpu

N_DEV = 8
B, SQ, SKV, DM = 2, 512, 512, 768
HQ_PER = 8
DH = 64
DP = HQ_PER * DH
BLK = 64
ROWS = B * SQ

PARTS = (
    {"base": 0, "size": 384, "order": "xyz"},
    {"base": 384, "size": 320, "order": "yzx"},
    {"base": 704, "size": 320, "order": "zxy"},
)


def kernel(x, Wq, K_ext, V_ext, Wo):
    my = lax.axis_index("i")
    Kh = lax.dynamic_slice_in_dim(K_ext, my * HQ_PER, HQ_PER, axis=2)
    Vh = lax.dynamic_slice_in_dim(V_ext, my * HQ_PER, HQ_PER, axis=2)
    Kt = jnp.transpose(Kh, (0, 2, 1, 3))
    Vt = jnp.transpose(Vh, (0, 2, 1, 3))
    xf = x.reshape(ROWS, DM)

    def body(x_ref, wq_ref, k_ref, v_ref, wo_ref, out_ref,
             ctx_ref, comm_a, comm_b, comm_c,
             stage_a, stage_b, stage_c, ag_a, ag_b, ag_c,
             send_sems, recv_sems):
        my_pos = lax.axis_index("i")
        m4 = my_pos % 4
        bit = {
            "x": jnp.where((m4 == 1) | (m4 == 2), 1, 0),
            "y": jnp.where(m4 >= 2, 1, 0),
            "z": jnp.where(my_pos >= 4, 1, 0),
        }
        ptn = {
            "x": my_pos + 1 - 2 * (my_pos % 2),
            "y": 4 * (my_pos // 4) + 3 - m4,
            "z": (my_pos + 4) % N_DEV,
        }

        barrier_sem = pltpu.get_barrier_semaphore()
        for d in "xyz":
            pl.semaphore_signal(
                barrier_sem, inc=1,
                device_id=(ptn[d],), device_id_type=pl.DeviceIdType.MESH,
            )
        pl.semaphore_wait(barrier_sem, 3)

        qb = lax.broadcasted_iota(jnp.int32, (SQ, SKV), 0) // BLK
        kb = lax.broadcasted_iota(jnp.int32, (SQ, SKV), 1) // BLK
        mask = kb <= qb

        q_all = jnp.dot(x_ref[...], wq_ref[...],
                        preferred_element_type=jnp.float32)
        for b in range(B):
            for h in range(HQ_PER):
                q = q_all[b * SQ:(b + 1) * SQ, h * DH:(h + 1) * DH]
                s = lax.dot_general(
                    q, k_ref[b, h], (((1,), (1,)), ((), ())),
                    preferred_element_type=jnp.float32) * 0.125
                s = jnp.where(mask, s, -1e9)
                m = jnp.max(s, axis=-1, keepdims=True)
                w = jnp.exp(s - m)
                w = w / jnp.sum(w, axis=-1, keepdims=True)
                ctx_ref[b * SQ:(b + 1) * SQ, h * DH:(h + 1) * DH] = jnp.dot(
                    w, v_ref[b, h], preferred_element_type=jnp.float32)
        out_ref[...] = jnp.dot(ctx_ref[...], wo_ref[...],
                               preferred_element_type=jnp.float32)

        comms = {0: comm_a, 1: comm_b, 2: comm_c}
        stages = {0: stage_a, 1: stage_b, 2: stage_c}
        ags = {0: ag_a, 1: ag_b, 2: ag_c}
        cur_off = [jnp.int32(p["base"]) for p in PARTS]

        for k in range(3):
            slot = k % 2
            started = []
            for pi, p in enumerate(PARTS):
                h = (p["size"] >> k) // 2
                d = p["order"][k]
                send_off = cur_off[pi] + (1 - bit[d]) * h
                keep_off = cur_off[pi] + bit[d] * h
                hmax = p["size"] // 2
                stages[pi][pl.ds(0, h), :] = out_ref[
                    pl.ds(send_off, h), :].astype(jnp.bfloat16)
                rdma = pltpu.make_async_remote_copy(
                    src_ref=stages[pi].at[pl.ds(0, h), :],
                    dst_ref=comms[pi].at[pl.ds(slot * hmax, h), :],
                    send_sem=send_sems.at[2 * pi + slot],
                    recv_sem=recv_sems.at[2 * pi + slot],
                    device_id=(ptn[d],),
                    device_id_type=pl.DeviceIdType.MESH,
                )
                rdma.start()
                started.append((rdma, pi, keep_off, h, slot * hmax))
                cur_off[pi] = keep_off
            for rdma, pi, keep_off, h, coff in started:
                rdma.wait()
                out_ref[pl.ds(keep_off, h), :] = (
                    out_ref[pl.ds(keep_off, h), :]
                    + comms[pi][pl.ds(coff, h), :].astype(jnp.float32))

        for pi, p in enumerate(PARTS):
            g0 = p["size"] >> 3
            ags[pi][pl.ds(cur_off[pi] - p["base"], g0), :] = out_ref[
                pl.ds(cur_off[pi], g0), :].astype(jnp.bfloat16)
        for k in range(3):
            slot = (3 + k) % 2
            started = []
            for pi, p in enumerate(PARTS):
                g = (p["size"] >> 3) << k
                d = p["order"][2 - k]
                loc = cur_off[pi] - p["base"]
                rdma = pltpu.make_async_remote_copy(
                    src_ref=ags[pi].at[pl.ds(loc, g), :],
                    dst_ref=ags[pi].at[pl.ds(loc, g), :],
                    send_sem=send_sems.at[2 * pi + slot],
                    recv_sem=recv_sems.at[2 * pi + slot],
                    device_id=(ptn[d],),
                    device_id_type=pl.DeviceIdType.MESH,
                )
                rdma.start()
                sib = cur_off[pi] + (1 - 2 * bit[d]) * g
                started.append((rdma, pi, sib, g))
                cur_off[pi] = cur_off[pi] - bit[d] * g
            for rdma, pi, sib, g in started:
                rdma.wait()
                out_ref[pl.ds(sib, g), :] = ags[pi][
                    pl.ds(sib - PARTS[pi]["base"], g), :].astype(jnp.float32)

    out2d = pl.pallas_call(
        body,
        out_shape=jax.ShapeDtypeStruct((ROWS, DM), jnp.float32),
        in_specs=[pl.BlockSpec(memory_space=pltpu.VMEM)] * 5,
        out_specs=pl.BlockSpec(memory_space=pltpu.VMEM),
        scratch_shapes=[
            pltpu.VMEM((ROWS, DP), jnp.float32),
            pltpu.VMEM((2 * 192, DM), jnp.bfloat16),
            pltpu.VMEM((2 * 160, DM), jnp.bfloat16),
            pltpu.VMEM((2 * 160, DM), jnp.bfloat16),
            pltpu.VMEM((192, DM), jnp.bfloat16),
            pltpu.VMEM((160, DM), jnp.bfloat16),
            pltpu.VMEM((160, DM), jnp.bfloat16),
            pltpu.VMEM((384, DM), jnp.bfloat16),
            pltpu.VMEM((320, DM), jnp.bfloat16),
            pltpu.VMEM((320, DM), jnp.bfloat16),
            pltpu.SemaphoreType.DMA((6,)),
            pltpu.SemaphoreType.DMA((6,)),
        ],
        compiler_params=pltpu.CompilerParams(collective_id=0),
    )(xf, Wq, Kt, Vt, Wo)
    return out2d.reshape(B, SQ, DM)
